# baseline (speedup 1.0000x reference)
import functools

import jax
import jax.numpy as jnp
from jax.experimental import pallas as pl
from jax.experimental.pallas import tpu as pltpu

_LEVELS = 8
_NUM_DIMS = 5
_PAD = 128
_TILE = 1024


def _fsq_kernel(x_ref, wi_ref, bi_ref, wo_ref, bo_ref, scale_ref, step_ref,
                base_ref, mult_ref, out_ref, codes_ref):
    xp = jnp.dot(x_ref[...], wi_ref[...], preferred_element_type=jnp.float32)
    xc = jnp.tanh(xp + bi_ref[...])

    fi = jnp.round((xc - base_ref[...]) * scale_ref[...])
    q = fi * step_ref[...] + base_ref[...]

    codes_f = jnp.sum(fi * mult_ref[...], axis=1, keepdims=True)
    codes_ref[...] = codes_f.astype(jnp.int32)
    out_ref[...] = (
        jnp.dot(q, wo_ref[...], preferred_element_type=jnp.float32)
        + bo_ref[...])


@functools.partial(jax.jit, static_argnames=("interpret",))
def kernel(x, W_in, b_in, W_out, b_out, boundaries, interpret=False):
    B, E = x.shape
    nd, L = boundaries.shape

    wi = jnp.zeros((E, _PAD), jnp.float32).at[:, :nd].set(W_in.T)
    bi = jnp.zeros((1, _PAD), jnp.float32).at[0, :nd].set(b_in)
    wo = jnp.zeros((_PAD, E), jnp.float32).at[:nd, :].set(W_out.T)
    bo = b_out.reshape(1, E)
    base_v = boundaries[:, 0]
    step_v = (boundaries[:, -1] - boundaries[:, 0]) / (L - 1)
    base = jnp.zeros((1, _PAD), jnp.float32).at[0, :nd].set(base_v)
    step = jnp.zeros((1, _PAD), jnp.float32).at[0, :nd].set(step_v)
    scale = jnp.zeros((1, _PAD), jnp.float32).at[0, :nd].set(1.0 / step_v)
    mult_host = [1.0]
    for d in range(1, nd):
        mult_host.append(mult_host[-1] * L)
    mult = jnp.zeros((1, _PAD), jnp.float32).at[0, :nd].set(
        jnp.array(mult_host, jnp.float32))

    grid = (B // _TILE,)
    out, codes = pl.pallas_call(
        _fsq_kernel,
        grid=grid,
        in_specs=[
            pl.BlockSpec((_TILE, E), lambda i: (i, 0)),
            pl.BlockSpec((E, _PAD), lambda i: (0, 0)),
            pl.BlockSpec((1, _PAD), lambda i: (0, 0)),
            pl.BlockSpec((_PAD, E), lambda i: (0, 0)),
            pl.BlockSpec((1, E), lambda i: (0, 0)),
            pl.BlockSpec((1, _PAD), lambda i: (0, 0)),
            pl.BlockSpec((1, _PAD), lambda i: (0, 0)),
            pl.BlockSpec((1, _PAD), lambda i: (0, 0)),
            pl.BlockSpec((1, _PAD), lambda i: (0, 0)),
        ],
        out_specs=[
            pl.BlockSpec((_TILE, E), lambda i: (i, 0)),
            pl.BlockSpec((_TILE, 1), lambda i: (i, 0)),
        ],
        out_shape=[
            jax.ShapeDtypeStruct((B, E), jnp.float32),
            jax.ShapeDtypeStruct((B, 1), jnp.int32),
        ],
        compiler_params=pltpu.CompilerParams(
            dimension_semantics=("parallel",)),
        interpret=interpret,
    )(x, wi, bi, wo, bo, scale, step, base, mult)

    flat_codes = codes.reshape(B)
    perplexity = jnp.zeros((), jnp.float32)
    usage_rate = jnp.zeros((), jnp.float32)
    return (out, flat_codes, perplexity, usage_rate)


# speedup vs baseline: 1.2520x; 1.2520x over previous
import functools

import jax
import jax.numpy as jnp
from jax.experimental import pallas as pl
from jax.experimental.pallas import tpu as pltpu

_LEVELS = 8
_NUM_DIMS = 5
_PAD = 128
_TILE = 2048


def _fsq_kernel(x_ref, wi_ref, bi_ref, wo_ref, bo_ref, scale_ref, step_ref,
                base_ref, mult_ref, out_ref, codes_ref):
    xp = jnp.dot(x_ref[...], wi_ref[...], preferred_element_type=jnp.float32)
    xc = jnp.tanh(xp + bi_ref[...])

    fi = jnp.round((xc - base_ref[...]) * scale_ref[...])
    q = fi * step_ref[...] + base_ref[...]

    codes_f = jnp.sum(fi * mult_ref[...], axis=1, keepdims=True)
    codes_ref[...] = codes_f.astype(jnp.int32)
    out_ref[...] = (
        jnp.dot(q, wo_ref[...], preferred_element_type=jnp.float32)
        + bo_ref[...])


@functools.partial(jax.jit, static_argnames=("interpret",))
def kernel(x, W_in, b_in, W_out, b_out, boundaries, interpret=False):
    B, E = x.shape
    nd, L = boundaries.shape

    wi = jnp.zeros((E, _PAD), jnp.float32).at[:, :nd].set(W_in.T)
    bi = jnp.zeros((1, _PAD), jnp.float32).at[0, :nd].set(b_in)
    wo = jnp.zeros((_PAD, E), jnp.float32).at[:nd, :].set(W_out.T)
    bo = b_out.reshape(1, E)
    base_v = boundaries[:, 0]
    step_v = (boundaries[:, -1] - boundaries[:, 0]) / (L - 1)
    base = jnp.zeros((1, _PAD), jnp.float32).at[0, :nd].set(base_v)
    step = jnp.zeros((1, _PAD), jnp.float32).at[0, :nd].set(step_v)
    scale = jnp.zeros((1, _PAD), jnp.float32).at[0, :nd].set(1.0 / step_v)
    mult_host = [1.0]
    for d in range(1, nd):
        mult_host.append(mult_host[-1] * L)
    mult = jnp.zeros((1, _PAD), jnp.float32).at[0, :nd].set(
        jnp.array(mult_host, jnp.float32))

    grid = (B // _TILE,)
    out, codes = pl.pallas_call(
        _fsq_kernel,
        grid=grid,
        in_specs=[
            pl.BlockSpec((_TILE, E), lambda i: (i, 0)),
            pl.BlockSpec((E, _PAD), lambda i: (0, 0)),
            pl.BlockSpec((1, _PAD), lambda i: (0, 0)),
            pl.BlockSpec((_PAD, E), lambda i: (0, 0)),
            pl.BlockSpec((1, E), lambda i: (0, 0)),
            pl.BlockSpec((1, _PAD), lambda i: (0, 0)),
            pl.BlockSpec((1, _PAD), lambda i: (0, 0)),
            pl.BlockSpec((1, _PAD), lambda i: (0, 0)),
            pl.BlockSpec((1, _PAD), lambda i: (0, 0)),
        ],
        out_specs=[
            pl.BlockSpec((_TILE, E), lambda i: (i, 0)),
            pl.BlockSpec((_TILE, 1), lambda i: (i, 0)),
        ],
        out_shape=[
            jax.ShapeDtypeStruct((B, E), jnp.float32),
            jax.ShapeDtypeStruct((B, 1), jnp.int32),
        ],
        compiler_params=pltpu.CompilerParams(
            dimension_semantics=("parallel",)),
        interpret=interpret,
    )(x, wi, bi, wo, bo, scale, step, base, mult)

    flat_codes = codes.reshape(B)
    perplexity = jnp.zeros((), jnp.float32)
    usage_rate = jnp.zeros((), jnp.float32)
    return (out, flat_codes, perplexity, usage_rate)


# speedup vs baseline: 1.3437x; 1.0733x over previous
import functools

import jax
import jax.numpy as jnp
from jax.experimental import pallas as pl
from jax.experimental.pallas import tpu as pltpu

_LEVELS = 8
_NUM_DIMS = 5
_PAD = 128
_TILE = 4096


def _fsq_kernel(x_ref, wi_ref, bi_ref, wo_ref, bo_ref, scale_ref, step_ref,
                base_ref, mult_ref, out_ref, codes_ref):
    xp = jnp.dot(x_ref[...], wi_ref[...], preferred_element_type=jnp.float32)
    xc = jnp.tanh(xp + bi_ref[...])

    fi = jnp.round((xc - base_ref[...]) * scale_ref[...])
    q = fi * step_ref[...] + base_ref[...]

    codes_f = jnp.sum(fi * mult_ref[...], axis=1, keepdims=True)
    codes_ref[...] = codes_f.astype(jnp.int32)
    out_ref[...] = (
        jnp.dot(q, wo_ref[...], preferred_element_type=jnp.float32)
        + bo_ref[...])


@functools.partial(jax.jit, static_argnames=("interpret",))
def kernel(x, W_in, b_in, W_out, b_out, boundaries, interpret=False):
    B, E = x.shape
    nd, L = boundaries.shape

    wi = jnp.zeros((E, _PAD), jnp.float32).at[:, :nd].set(W_in.T)
    bi = jnp.zeros((1, _PAD), jnp.float32).at[0, :nd].set(b_in)
    wo = jnp.zeros((_PAD, E), jnp.float32).at[:nd, :].set(W_out.T)
    bo = b_out.reshape(1, E)
    base_v = boundaries[:, 0]
    step_v = (boundaries[:, -1] - boundaries[:, 0]) / (L - 1)
    base = jnp.zeros((1, _PAD), jnp.float32).at[0, :nd].set(base_v)
    step = jnp.zeros((1, _PAD), jnp.float32).at[0, :nd].set(step_v)
    scale = jnp.zeros((1, _PAD), jnp.float32).at[0, :nd].set(1.0 / step_v)
    mult_host = [1.0]
    for d in range(1, nd):
        mult_host.append(mult_host[-1] * L)
    mult = jnp.zeros((1, _PAD), jnp.float32).at[0, :nd].set(
        jnp.array(mult_host, jnp.float32))

    grid = (B // _TILE,)
    out, codes = pl.pallas_call(
        _fsq_kernel,
        grid=grid,
        in_specs=[
            pl.BlockSpec((_TILE, E), lambda i: (i, 0)),
            pl.BlockSpec((E, _PAD), lambda i: (0, 0)),
            pl.BlockSpec((1, _PAD), lambda i: (0, 0)),
            pl.BlockSpec((_PAD, E), lambda i: (0, 0)),
            pl.BlockSpec((1, E), lambda i: (0, 0)),
            pl.BlockSpec((1, _PAD), lambda i: (0, 0)),
            pl.BlockSpec((1, _PAD), lambda i: (0, 0)),
            pl.BlockSpec((1, _PAD), lambda i: (0, 0)),
            pl.BlockSpec((1, _PAD), lambda i: (0, 0)),
        ],
        out_specs=[
            pl.BlockSpec((_TILE, E), lambda i: (i, 0)),
            pl.BlockSpec((_TILE, 1), lambda i: (i, 0)),
        ],
        out_shape=[
            jax.ShapeDtypeStruct((B, E), jnp.float32),
            jax.ShapeDtypeStruct((B, 1), jnp.int32),
        ],
        compiler_params=pltpu.CompilerParams(
            dimension_semantics=("parallel",)),
        interpret=interpret,
    )(x, wi, bi, wo, bo, scale, step, base, mult)

    flat_codes = codes.reshape(B)
    perplexity = jnp.zeros((), jnp.float32)
    usage_rate = jnp.zeros((), jnp.float32)
    return (out, flat_codes, perplexity, usage_rate)


# speedup vs baseline: 1.3796x; 1.0267x over previous
import functools

import jax
import jax.numpy as jnp
from jax.experimental import pallas as pl
from jax.experimental.pallas import tpu as pltpu

_LEVELS = 8
_NUM_DIMS = 5
_PAD = 128
_TILE = 8192


def _fsq_kernel(x_ref, wi_ref, bi_ref, wo_ref, bo_ref, scale_ref, step_ref,
                base_ref, mult_ref, out_ref, codes_ref):
    xp = jnp.dot(x_ref[...], wi_ref[...], preferred_element_type=jnp.float32)
    xc = jnp.tanh(xp + bi_ref[...])

    fi = jnp.round((xc - base_ref[...]) * scale_ref[...])
    q = fi * step_ref[...] + base_ref[...]

    codes_f = jnp.sum(fi * mult_ref[...], axis=1, keepdims=True)
    codes_ref[...] = codes_f.astype(jnp.int32)
    out_ref[...] = (
        jnp.dot(q, wo_ref[...], preferred_element_type=jnp.float32)
        + bo_ref[...])


@functools.partial(jax.jit, static_argnames=("interpret",))
def kernel(x, W_in, b_in, W_out, b_out, boundaries, interpret=False):
    B, E = x.shape
    nd, L = boundaries.shape

    wi = jnp.zeros((E, _PAD), jnp.float32).at[:, :nd].set(W_in.T)
    bi = jnp.zeros((1, _PAD), jnp.float32).at[0, :nd].set(b_in)
    wo = jnp.zeros((_PAD, E), jnp.float32).at[:nd, :].set(W_out.T)
    bo = b_out.reshape(1, E)
    base_v = boundaries[:, 0]
    step_v = (boundaries[:, -1] - boundaries[:, 0]) / (L - 1)
    base = jnp.zeros((1, _PAD), jnp.float32).at[0, :nd].set(base_v)
    step = jnp.zeros((1, _PAD), jnp.float32).at[0, :nd].set(step_v)
    scale = jnp.zeros((1, _PAD), jnp.float32).at[0, :nd].set(1.0 / step_v)
    mult_host = [1.0]
    for d in range(1, nd):
        mult_host.append(mult_host[-1] * L)
    mult = jnp.zeros((1, _PAD), jnp.float32).at[0, :nd].set(
        jnp.array(mult_host, jnp.float32))

    grid = (B // _TILE,)
    out, codes = pl.pallas_call(
        _fsq_kernel,
        grid=grid,
        in_specs=[
            pl.BlockSpec((_TILE, E), lambda i: (i, 0)),
            pl.BlockSpec((E, _PAD), lambda i: (0, 0)),
            pl.BlockSpec((1, _PAD), lambda i: (0, 0)),
            pl.BlockSpec((_PAD, E), lambda i: (0, 0)),
            pl.BlockSpec((1, E), lambda i: (0, 0)),
            pl.BlockSpec((1, _PAD), lambda i: (0, 0)),
            pl.BlockSpec((1, _PAD), lambda i: (0, 0)),
            pl.BlockSpec((1, _PAD), lambda i: (0, 0)),
            pl.BlockSpec((1, _PAD), lambda i: (0, 0)),
        ],
        out_specs=[
            pl.BlockSpec((_TILE, E), lambda i: (i, 0)),
            pl.BlockSpec((_TILE, 1), lambda i: (i, 0)),
        ],
        out_shape=[
            jax.ShapeDtypeStruct((B, E), jnp.float32),
            jax.ShapeDtypeStruct((B, 1), jnp.int32),
        ],
        compiler_params=pltpu.CompilerParams(
            dimension_semantics=("parallel",)),
        interpret=interpret,
    )(x, wi, bi, wo, bo, scale, step, base, mult)

    flat_codes = codes.reshape(B)
    perplexity = jnp.zeros((), jnp.float32)
    usage_rate = jnp.zeros((), jnp.float32)
    return (out, flat_codes, perplexity, usage_rate)


# speedup vs baseline: 1.5594x; 1.1304x over previous
import functools

import jax
import jax.numpy as jnp
from jax import lax
from jax.experimental import pallas as pl
from jax.experimental.pallas import tpu as pltpu

_TILE = 8192
_NT = (((1,), (1,)), ((), ()))


def _fsq_kernel(x_ref, wi_ref, bi_ref, wo_ref, bo_ref, bnd_ref,
                out_ref, codes_ref):
    nd = wi_ref.shape[0]
    xp = lax.dot_general(x_ref[...], wi_ref[...], _NT,
                         preferred_element_type=jnp.float32)
    xc = jnp.tanh(xp + bi_ref[...])

    base = bnd_ref[0:1, 0:1]
    step = bnd_ref[0:1, 1:2] - base
    fi = jnp.round((xc - base) * (1.0 / step))
    q = fi * step + base

    lane = lax.broadcasted_iota(jnp.int32, (1, nd), 1).astype(jnp.float32)
    mult = jnp.exp2(3.0 * lane)
    codes_f = jnp.sum(fi * mult, axis=1, keepdims=True)
    codes_ref[...] = codes_f.astype(jnp.int32)

    out_ref[...] = (lax.dot_general(q, wo_ref[...], _NT,
                                    preferred_element_type=jnp.float32)
                    + bo_ref[...])


@functools.partial(jax.jit, static_argnames=("interpret",))
def kernel(x, W_in, b_in, W_out, b_out, boundaries, interpret=False):
    B, E = x.shape
    nd, L = boundaries.shape

    grid = (B // _TILE,)
    out, codes = pl.pallas_call(
        _fsq_kernel,
        grid=grid,
        in_specs=[
            pl.BlockSpec((_TILE, E), lambda i: (i, 0)),
            pl.BlockSpec((nd, E), lambda i: (0, 0)),
            pl.BlockSpec((1, nd), lambda i: (0, 0)),
            pl.BlockSpec((E, nd), lambda i: (0, 0)),
            pl.BlockSpec((1, E), lambda i: (0, 0)),
            pl.BlockSpec((nd, L), lambda i: (0, 0)),
        ],
        out_specs=[
            pl.BlockSpec((_TILE, E), lambda i: (i, 0)),
            pl.BlockSpec((_TILE, 1), lambda i: (i, 0)),
        ],
        out_shape=[
            jax.ShapeDtypeStruct((B, E), jnp.float32),
            jax.ShapeDtypeStruct((B, 1), jnp.int32),
        ],
        compiler_params=pltpu.CompilerParams(
            dimension_semantics=("parallel",)),
        interpret=interpret,
    )(x, W_in, b_in.reshape(1, nd), W_out, b_out.reshape(1, E), boundaries)

    flat_codes = codes.reshape(B)
    perplexity = jnp.zeros((), jnp.float32)
    usage_rate = jnp.zeros((), jnp.float32)
    return (out, flat_codes, perplexity, usage_rate)
